# Initial kernel scaffold; baseline (speedup 1.0000x reference)
import jax
import jax.numpy as jnp
from jax.experimental import pallas as pl


def kernel(x, W_le, W1, W2, W3, W4, W_last, W_lin1, bn_g, bn_b, W_lin2, b_lin2):
    raise NotImplementedError("write your pallas kernel here")


# speedup vs baseline: 9.6874x; 9.6874x over previous
import functools

import jax
import jax.numpy as jnp
from jax import lax
from jax.experimental import pallas as pl
from jax.experimental.pallas import tpu as pltpu
from jax.experimental.pallas import tpu_sc as plsc

_NC = 2
_NS = 16
_NW = _NC * _NS
_K = 16
_INF = 3.0e38


def _level_body(N, S, feat_ref, xyz_ref, xyzT_ref, wsum_ref, wb_ref,
                g_ref, hc_ref, idx_ref):
    b = pl.program_id(0)
    feat = feat_ref[0]
    xyz = xyz_ref[0]
    xyzT = xyzT_ref[0]

    n_col = jnp.sum(xyz * xyz, axis=1, keepdims=True)
    n_row = jnp.sum(xyzT * xyzT, axis=0, keepdims=True)
    cross = jnp.dot(xyz[:S, :], xyzT,
                    preferred_element_type=jnp.float32)
    d2 = n_col[:S, :] + n_row - 2.0 * cross

    iota_n = lax.broadcasted_iota(jnp.int32, (S, N), 1)
    cols = []
    for _ in range(_K):
        m = jnp.min(d2, axis=1, keepdims=True)
        am = jnp.min(jnp.where(d2 <= m, iota_n, N), axis=1, keepdims=True)
        cols.append(am)
        d2 = jnp.where(iota_n == am, _INF, d2)
    idx_ref[0] = jnp.concatenate(cols, axis=1) + b * N

    g_ref[0] = jnp.dot(feat, wsum_ref[...],
                       preferred_element_type=jnp.float32)
    hc_ref[0] = jnp.dot(feat[:S, :], wb_ref[...],
                        preferred_element_type=jnp.float32)


def _make_level(B, N, S, C, D):
    body = functools.partial(_level_body, N, S)
    return pl.pallas_call(
        body,
        grid=(B,),
        in_specs=[
            pl.BlockSpec((1, N, C), lambda b: (b, 0, 0)),
            pl.BlockSpec((1, N, 3), lambda b: (b, 0, 0)),
            pl.BlockSpec((1, 3, N), lambda b: (b, 0, 0)),
            pl.BlockSpec((C, D), lambda b: (0, 0)),
            pl.BlockSpec((C, D), lambda b: (0, 0)),
        ],
        out_specs=[
            pl.BlockSpec((1, N, D), lambda b: (b, 0, 0)),
            pl.BlockSpec((1, S, D), lambda b: (b, 0, 0)),
            pl.BlockSpec((1, S, _K), lambda b: (b, 0, 0)),
        ],
        out_shape=[
            jax.ShapeDtypeStruct((B, N, D), jnp.float32),
            jax.ShapeDtypeStruct((B, S, D), jnp.float32),
            jax.ShapeDtypeStruct((B, S, _K), jnp.int32),
        ],
    )


def _fold_body(m_ref, hc_ref, out_ref):
    out_ref[...] = jax.nn.gelu(m_ref[...] - hc_ref[...])


def _make_fold(B, S, D):
    return pl.pallas_call(
        _fold_body,
        grid=(B,),
        in_specs=[
            pl.BlockSpec((1, S, D), lambda b: (b, 0, 0)),
            pl.BlockSpec((1, S, D), lambda b: (b, 0, 0)),
        ],
        out_specs=pl.BlockSpec((1, S, D), lambda b: (b, 0, 0)),
        out_shape=jax.ShapeDtypeStruct((B, S, D), jnp.float32),
    )


def _first_fold_body(x_ref, wle_ref, out_ref):
    out_ref[...] = jax.nn.gelu(
        jnp.dot(x_ref[...], wle_ref[...], preferred_element_type=jnp.float32))


def _make_fold_first(B, N):
    return pl.pallas_call(
        _first_fold_body,
        grid=(B,),
        in_specs=[
            pl.BlockSpec((1, N, 3), lambda b: (b, 0, 0)),
            pl.BlockSpec((3, 32), lambda b: (0, 0)),
        ],
        out_specs=pl.BlockSpec((1, N, 32), lambda b: (b, 0, 0)),
        out_shape=jax.ShapeDtypeStruct((B, N, 32), jnp.float32),
    )


def _make_sc_gathermax(R, NG, D):
    gpw = NG // _NW
    gc = max(1, min(8, 2048 // D))
    while gpw % gc:
        gc //= 2
    nchunks = gpw // gc
    rows = gc * _K
    mesh = plsc.VectorSubcoreMesh(core_axis_name="c", subcore_axis_name="s")

    @functools.partial(
        pl.kernel,
        out_type=jax.ShapeDtypeStruct((NG, D), jnp.float32),
        mesh=mesh,
        compiler_params=pltpu.CompilerParams(use_tc_tiling_on_sc=False),
        scratch_types=[
            pltpu.VMEM((rows,), jnp.int32),
            pltpu.VMEM((rows, D), jnp.float32),
            pltpu.VMEM((gc, D), jnp.float32),
            pltpu.SemaphoreType.DMA,
        ],
    )
    def sck(table_hbm, idx_hbm, out_hbm, idx_v, rows_v, out_v, sem):
        wid = lax.axis_index("s") * _NC + lax.axis_index("c")
        base_g = wid * gpw

        def chunk(i, carry):
            g0 = base_g + i * gc
            pltpu.sync_copy(idx_hbm.at[pl.ds(g0 * _K, rows)], idx_v)
            pltpu.async_copy(table_hbm.at[idx_v], rows_v, sem).wait()
            for g in range(gc):
                for c in range(D // 16):
                    sl = pl.ds(c * 16, 16)
                    acc = rows_v[g * _K, sl]
                    for r in range(1, _K):
                        acc = jnp.maximum(acc, rows_v[g * _K + r, sl])
                    out_v[g, sl] = acc
            pltpu.sync_copy(out_v, out_hbm.at[pl.ds(g0, gc)])
            return carry

        lax.fori_loop(0, nchunks, chunk, 0)

    return sck


def _tail_body(B, S, m_ref, hc_ref, wlast_ref, wlin1_ref, bng_ref, bnb_ref,
               wlin2_ref, blin2_ref, out_ref):
    feat = jax.nn.gelu(m_ref[...] - hc_ref[...])
    h = jax.nn.gelu(jnp.dot(feat, wlast_ref[...],
                            preferred_element_type=jnp.float32))
    h3 = jnp.reshape(h, (B, S, h.shape[-1]))
    hmax = jnp.max(h3, axis=1)
    havg = jnp.mean(h3, axis=1)
    g = jnp.concatenate([hmax, havg], axis=-1)
    g = jnp.dot(g, wlin1_ref[...], preferred_element_type=jnp.float32)
    g = g * bng_ref[...] + bnb_ref[...]
    g = jax.nn.gelu(g)
    out_ref[...] = (jnp.dot(g, wlin2_ref[...],
                            preferred_element_type=jnp.float32)
                    + blin2_ref[...])


def _make_tail(B, S):
    return pl.pallas_call(
        functools.partial(_tail_body, B, S),
        out_shape=jax.ShapeDtypeStruct((B, 40), jnp.float32),
    )


def kernel(x, W_le, W1, W2, W3, W4, W_last, W_lin1, bn_g, bn_b,
           W_lin2, b_lin2):
    B, N, _ = x.shape
    levels = [
        (N, N, 32, 64, W1),
        (N, N // 2, 64, 128, W2),
        (N // 2, N // 4, 128, 256, W3),
        (N // 4, N // 8, 256, 512, W4),
    ]
    xyz = x
    feat = _make_fold_first(B, N)(x, W_le)
    for li, (n, s, c, d, w) in enumerate(levels):
        wa, wb = w[:c, :], w[c:, :]
        wsum = wa + wb
        xyz_l = xyz[:, :n, :]
        xyzT_l = jnp.transpose(xyz_l, (0, 2, 1))
        if li > 0:
            feat = _make_fold(B, n, c)(m_prev, hc_prev)
        g, hc, idx = _make_level(B, n, s, c, d)(
            feat, xyz_l, xyzT_l, wsum, wb)
        m = _make_sc_gathermax(B * n, B * s, d)(
            jnp.reshape(g, (B * n, d)),
            jnp.reshape(idx, (B * s * _K,)))
        m_prev = jnp.reshape(m, (B, s, d))
        hc_prev = hc
        xyz = xyz_l[:, :s, :]

    S4 = N // 8
    out = _make_tail(B, S4)(
        jnp.reshape(m_prev, (B * S4, 512)),
        jnp.reshape(hc_prev, (B * S4, 512)),
        W_last, W_lin1,
        jnp.reshape(bn_g, (1, -1)), jnp.reshape(bn_b, (1, -1)),
        W_lin2, jnp.reshape(b_lin2, (1, -1)))
    return out


# speedup vs baseline: 12.3344x; 1.2732x over previous
import functools

import jax
import jax.numpy as jnp
from jax import lax
from jax.experimental import pallas as pl
from jax.experimental.pallas import tpu as pltpu
from jax.experimental.pallas import tpu_sc as plsc

_NC = 2
_NS = 16
_NW = _NC * _NS
_K = 16
_INF = 3.0e38


def _topk_body(N, S, xyz_ref, xyzT_ref, idx_ref):
    b = pl.program_id(0)
    xyz = xyz_ref[0]
    xyzT = xyzT_ref[0]

    n_col = jnp.sum(xyz * xyz, axis=1, keepdims=True)
    n_row = jnp.sum(xyzT * xyzT, axis=0, keepdims=True)
    cross = jnp.dot(xyz[:S, :], xyzT,
                    preferred_element_type=jnp.float32)
    d2 = n_col[:S, :] + n_row - 2.0 * cross

    iota_n = lax.broadcasted_iota(jnp.int32, (S, N), 1)
    cols = []
    for _ in range(_K):
        m = jnp.min(d2, axis=1, keepdims=True)
        am = jnp.min(jnp.where(d2 <= m, iota_n, N), axis=1, keepdims=True)
        cols.append(am)
        d2 = jnp.where(iota_n == am, _INF, d2)
    idx_ref[0] = jnp.concatenate(cols, axis=1) + b * N


def _make_topk(B, N, S):
    return pl.pallas_call(
        functools.partial(_topk_body, N, S),
        grid=(B,),
        in_specs=[
            pl.BlockSpec((1, N, 3), lambda b: (b, 0, 0)),
            pl.BlockSpec((1, 3, N), lambda b: (b, 0, 0)),
        ],
        out_specs=pl.BlockSpec((1, S, _K), lambda b: (b, 0, 0)),
        out_shape=jax.ShapeDtypeStruct((B, S, _K), jnp.int32),
    )


def _make_sc_gather(R, NR, C):
    rpw = NR // _NW
    rc = 128
    while rpw % rc:
        rc //= 2
    nchunks = rpw // rc
    mesh = plsc.VectorSubcoreMesh(core_axis_name="c", subcore_axis_name="s")

    @functools.partial(
        pl.kernel,
        out_type=jax.ShapeDtypeStruct((NR, C), jnp.float32),
        mesh=mesh,
        compiler_params=pltpu.CompilerParams(use_tc_tiling_on_sc=False),
        scratch_types=[
            pltpu.VMEM((rc,), jnp.int32),
            pltpu.VMEM((rc, C), jnp.float32),
            pltpu.SemaphoreType.DMA,
        ],
    )
    def sck(table_hbm, idx_hbm, out_hbm, idx_v, rows_v, sem):
        wid = lax.axis_index("s") * _NC + lax.axis_index("c")
        base_r = wid * rpw

        def chunk(i, carry):
            r0 = base_r + i * rc
            pltpu.sync_copy(idx_hbm.at[pl.ds(r0, rc)], idx_v)
            pltpu.async_copy(table_hbm.at[idx_v], rows_v, sem).wait()
            pltpu.sync_copy(rows_v, out_hbm.at[pl.ds(r0, rc)])
            return carry

        lax.fori_loop(0, nchunks, chunk, 0)

    return sck


def _conv_body(S, C, gath_ref, cent_ref, w_ref, out_ref):
    gath = jnp.reshape(gath_ref[0], (S, _K, C))
    cent = cent_ref[0][:S, :]
    edge = jnp.concatenate([gath, gath - cent[:, None, :]], axis=-1)
    edge2 = jnp.reshape(edge, (S * _K, 2 * C))
    t = jnp.dot(edge2, w_ref[...], preferred_element_type=jnp.float32)
    t3 = jnp.reshape(t, (S, _K, t.shape[-1]))
    out_ref[0] = jax.nn.gelu(jnp.max(t3, axis=1))


def _make_conv(B, N, S, C, D):
    return pl.pallas_call(
        functools.partial(_conv_body, S, C),
        grid=(B,),
        in_specs=[
            pl.BlockSpec((1, S * _K, C), lambda b: (b, 0, 0)),
            pl.BlockSpec((1, N, C), lambda b: (b, 0, 0)),
            pl.BlockSpec((2 * C, D), lambda b: (0, 0)),
        ],
        out_specs=pl.BlockSpec((1, S, D), lambda b: (b, 0, 0)),
        out_shape=jax.ShapeDtypeStruct((B, S, D), jnp.float32),
    )


def _head_body(x_ref, wle_ref, out_ref):
    out_ref[...] = jax.nn.gelu(
        jnp.dot(x_ref[...], wle_ref[...], preferred_element_type=jnp.float32))


def _make_head(B, N):
    return pl.pallas_call(
        _head_body,
        grid=(B,),
        in_specs=[
            pl.BlockSpec((1, N, 3), lambda b: (b, 0, 0)),
            pl.BlockSpec((3, 32), lambda b: (0, 0)),
        ],
        out_specs=pl.BlockSpec((1, N, 32), lambda b: (b, 0, 0)),
        out_shape=jax.ShapeDtypeStruct((B, N, 32), jnp.float32),
    )


def _tail_body(B, S, f_ref, wlast_ref, wlin1_ref, bng_ref, bnb_ref,
               wlin2_ref, blin2_ref, out_ref):
    h = jax.nn.gelu(jnp.dot(f_ref[...], wlast_ref[...],
                            preferred_element_type=jnp.float32))
    h3 = jnp.reshape(h, (B, S, h.shape[-1]))
    hmax = jnp.max(h3, axis=1)
    havg = jnp.mean(h3, axis=1)
    g = jnp.concatenate([hmax, havg], axis=-1)
    g = jnp.dot(g, wlin1_ref[...], preferred_element_type=jnp.float32)
    g = g * bng_ref[...] + bnb_ref[...]
    g = jax.nn.gelu(g)
    out_ref[...] = (jnp.dot(g, wlin2_ref[...],
                            preferred_element_type=jnp.float32)
                    + blin2_ref[...])


def _make_tail(B, S):
    return pl.pallas_call(
        functools.partial(_tail_body, B, S),
        out_shape=jax.ShapeDtypeStruct((B, 40), jnp.float32),
    )


def kernel(x, W_le, W1, W2, W3, W4, W_last, W_lin1, bn_g, bn_b,
           W_lin2, b_lin2):
    B, N, _ = x.shape
    levels = [
        (N, N, 32, 64, W1),
        (N, N // 2, 64, 128, W2),
        (N // 2, N // 4, 128, 256, W3),
        (N // 4, N // 8, 256, 512, W4),
    ]
    idxs = []
    xyz = x
    for (n, s, c, d, w) in levels:
        xyz_l = xyz[:, :n, :]
        idxs.append(_make_topk(B, n, s)(xyz_l, jnp.transpose(xyz_l, (0, 2, 1))))
        xyz = xyz_l[:, :s, :]

    feat = _make_head(B, N)(x, W_le)
    for li, (n, s, c, d, w) in enumerate(levels):
        gath = _make_sc_gather(B * n, B * s * _K, c)(
            jnp.reshape(feat, (B * n, c)),
            jnp.reshape(idxs[li], (B * s * _K,)))
        feat = _make_conv(B, n, s, c, d)(
            jnp.reshape(gath, (B, s * _K, c)), feat, w)

    S4 = N // 8
    return _make_tail(B, S4)(
        jnp.reshape(feat, (B * S4, 512)),
        W_last, W_lin1,
        jnp.reshape(bn_g, (1, -1)), jnp.reshape(bn_b, (1, -1)),
        W_lin2, jnp.reshape(b_lin2, (1, -1)))


# speedup vs baseline: 12.3843x; 1.0040x over previous
import functools

import jax
import jax.numpy as jnp
from jax import lax
from jax.experimental import pallas as pl
from jax.experimental.pallas import tpu as pltpu
from jax.experimental.pallas import tpu_sc as plsc

_NC = 2
_NS = 16
_NW = _NC * _NS
_K = 16
_INF = 3.0e38


def _topk_body(N, S, xyz_ref, xyzT_ref, idx_ref):
    b = pl.program_id(0)
    xyz = xyz_ref[0]
    xyzT = xyzT_ref[0]

    n_col = jnp.sum(xyz * xyz, axis=1, keepdims=True)
    n_row = jnp.sum(xyzT * xyzT, axis=0, keepdims=True)
    cross = jnp.dot(xyz[:S, :], xyzT,
                    preferred_element_type=jnp.float32)
    d2 = n_col[:S, :] + n_row - 2.0 * cross

    iota_n = lax.broadcasted_iota(jnp.int32, (S, N), 1)
    cols = []
    for _ in range(_K):
        m = jnp.min(d2, axis=1, keepdims=True)
        am = jnp.min(jnp.where(d2 <= m, iota_n, N), axis=1, keepdims=True)
        cols.append(am)
        d2 = jnp.where(iota_n == am, _INF, d2)
    idx_ref[0] = jnp.concatenate(cols, axis=1) + b * N


def _make_topk(B, N, S):
    return pl.pallas_call(
        functools.partial(_topk_body, N, S),
        grid=(B,),
        in_specs=[
            pl.BlockSpec((1, N, 3), lambda b: (b, 0, 0)),
            pl.BlockSpec((1, 3, N), lambda b: (b, 0, 0)),
        ],
        out_specs=pl.BlockSpec((1, S, _K), lambda b: (b, 0, 0)),
        out_shape=jax.ShapeDtypeStruct((B, S, _K), jnp.int32),
    )


def _make_sc_gather(R, NR, C):
    rpw = NR // _NW
    rc = 128
    while rpw % rc:
        rc //= 2
    nchunks = rpw // rc
    mesh = plsc.VectorSubcoreMesh(core_axis_name="c", subcore_axis_name="s")

    @functools.partial(
        pl.kernel,
        out_type=jax.ShapeDtypeStruct((NR, C), jnp.float32),
        mesh=mesh,
        compiler_params=pltpu.CompilerParams(use_tc_tiling_on_sc=False),
        scratch_types=[
            pltpu.VMEM((rpw,), jnp.int32),
            pltpu.VMEM((rc, C), jnp.float32),
            pltpu.VMEM((rc, C), jnp.float32),
            pltpu.SemaphoreType.DMA,
            pltpu.SemaphoreType.DMA,
        ],
    )
    def sck(table_hbm, idx_hbm, out_hbm, idx_v, rows0_v, rows1_v,
            sem0, sem1):
        wid = lax.axis_index("s") * _NC + lax.axis_index("c")
        base_r = wid * rpw
        bufs = (rows0_v, rows1_v)
        sems = (sem0, sem1)

        pltpu.sync_copy(idx_hbm.at[pl.ds(base_r, rpw)], idx_v)
        pltpu.async_copy(
            table_hbm.at[idx_v.at[pl.ds(0, rc)]], rows0_v, sem0)

        def pair(i, carry):
            for j in range(2):
                cur = i * 2 + j
                nxt = cur + 1

                @pl.when(nxt < nchunks)
                def _():
                    pltpu.async_copy(
                        table_hbm.at[idx_v.at[pl.ds(nxt * rc, rc)]],
                        bufs[1 - j], sems[1 - j])

                pltpu.make_async_copy(
                    table_hbm.at[idx_v.at[pl.ds(cur * rc, rc)]],
                    bufs[j], sems[j]).wait()
                pltpu.sync_copy(bufs[j],
                                out_hbm.at[pl.ds(base_r + cur * rc, rc)])
            return carry

        lax.fori_loop(0, nchunks // 2, pair, 0)

    return sck


def _conv_body(S, C, gath_ref, cent_ref, w_ref, out_ref):
    gath = jnp.reshape(gath_ref[0], (S, _K, C))
    cent = cent_ref[0][:S, :]
    edge = jnp.concatenate([gath, gath - cent[:, None, :]], axis=-1)
    edge2 = jnp.reshape(edge, (S * _K, 2 * C))
    t = jnp.dot(edge2, w_ref[...], preferred_element_type=jnp.float32)
    t3 = jnp.reshape(t, (S, _K, t.shape[-1]))
    out_ref[0] = jax.nn.gelu(jnp.max(t3, axis=1))


def _make_conv(B, N, S, C, D):
    return pl.pallas_call(
        functools.partial(_conv_body, S, C),
        grid=(B,),
        in_specs=[
            pl.BlockSpec((1, S * _K, C), lambda b: (b, 0, 0)),
            pl.BlockSpec((1, N, C), lambda b: (b, 0, 0)),
            pl.BlockSpec((2 * C, D), lambda b: (0, 0)),
        ],
        out_specs=pl.BlockSpec((1, S, D), lambda b: (b, 0, 0)),
        out_shape=jax.ShapeDtypeStruct((B, S, D), jnp.float32),
    )


def _head_body(x_ref, wle_ref, out_ref):
    out_ref[...] = jax.nn.gelu(
        jnp.dot(x_ref[...], wle_ref[...], preferred_element_type=jnp.float32))


def _make_head(B, N):
    return pl.pallas_call(
        _head_body,
        grid=(B,),
        in_specs=[
            pl.BlockSpec((1, N, 3), lambda b: (b, 0, 0)),
            pl.BlockSpec((3, 32), lambda b: (0, 0)),
        ],
        out_specs=pl.BlockSpec((1, N, 32), lambda b: (b, 0, 0)),
        out_shape=jax.ShapeDtypeStruct((B, N, 32), jnp.float32),
    )


def _tail_body(B, S, f_ref, wlast_ref, wlin1_ref, bng_ref, bnb_ref,
               wlin2_ref, blin2_ref, out_ref):
    h = jax.nn.gelu(jnp.dot(f_ref[...], wlast_ref[...],
                            preferred_element_type=jnp.float32))
    h3 = jnp.reshape(h, (B, S, h.shape[-1]))
    hmax = jnp.max(h3, axis=1)
    havg = jnp.mean(h3, axis=1)
    g = jnp.concatenate([hmax, havg], axis=-1)
    g = jnp.dot(g, wlin1_ref[...], preferred_element_type=jnp.float32)
    g = g * bng_ref[...] + bnb_ref[...]
    g = jax.nn.gelu(g)
    out_ref[...] = (jnp.dot(g, wlin2_ref[...],
                            preferred_element_type=jnp.float32)
                    + blin2_ref[...])


def _make_tail(B, S):
    return pl.pallas_call(
        functools.partial(_tail_body, B, S),
        out_shape=jax.ShapeDtypeStruct((B, 40), jnp.float32),
    )


def kernel(x, W_le, W1, W2, W3, W4, W_last, W_lin1, bn_g, bn_b,
           W_lin2, b_lin2):
    B, N, _ = x.shape
    levels = [
        (N, N, 32, 64, W1),
        (N, N // 2, 64, 128, W2),
        (N // 2, N // 4, 128, 256, W3),
        (N // 4, N // 8, 256, 512, W4),
    ]
    idxs = []
    xyz = x
    for (n, s, c, d, w) in levels:
        xyz_l = xyz[:, :n, :]
        idxs.append(_make_topk(B, n, s)(xyz_l, jnp.transpose(xyz_l, (0, 2, 1))))
        xyz = xyz_l[:, :s, :]

    feat = _make_head(B, N)(x, W_le)
    for li, (n, s, c, d, w) in enumerate(levels):
        gath = _make_sc_gather(B * n, B * s * _K, c)(
            jnp.reshape(feat, (B * n, c)),
            jnp.reshape(idxs[li], (B * s * _K,)))
        feat = _make_conv(B, n, s, c, d)(
            jnp.reshape(gath, (B, s * _K, c)), feat, w)

    S4 = N // 8
    return _make_tail(B, S4)(
        jnp.reshape(feat, (B * S4, 512)),
        W_last, W_lin1,
        jnp.reshape(bn_g, (1, -1)), jnp.reshape(bn_b, (1, -1)),
        W_lin2, jnp.reshape(b_lin2, (1, -1)))


# speedup vs baseline: 13.1616x; 1.0628x over previous
import functools

import jax
import jax.numpy as jnp
from jax import lax
from jax.experimental import pallas as pl
from jax.experimental.pallas import tpu as pltpu
from jax.experimental.pallas import tpu_sc as plsc

_NC = 2
_NS = 16
_NW = _NC * _NS
_K = 16
_INF = 3.0e38


def _topk_body(N, S, xyz_ref, xyzT_ref, idx_ref):
    b = pl.program_id(0)
    xyz = xyz_ref[0]
    xyzT = xyzT_ref[0]

    n_pt = jnp.sum(xyz * xyz, axis=1, keepdims=True)
    n_ct = jnp.sum(xyzT * xyzT, axis=0, keepdims=True)[:, :S]
    cross = jnp.dot(xyz, xyzT[:, :S],
                    preferred_element_type=jnp.float32)
    d2 = n_ct + n_pt - 2.0 * cross

    iota_n = lax.broadcasted_iota(jnp.int32, (N, S), 0)
    rows = []
    for _ in range(_K):
        m = jnp.min(d2, axis=0, keepdims=True)
        am = jnp.min(jnp.where(d2 <= m, iota_n, N), axis=0, keepdims=True)
        rows.append(am)
        d2 = jnp.where(iota_n == am, _INF, d2)
    idx_ref[0] = jnp.concatenate(rows, axis=0) + b * N


def _make_topk(B, N, S):
    return pl.pallas_call(
        functools.partial(_topk_body, N, S),
        grid=(B,),
        in_specs=[
            pl.BlockSpec((1, N, 3), lambda b: (b, 0, 0)),
            pl.BlockSpec((1, 3, N), lambda b: (b, 0, 0)),
        ],
        out_specs=pl.BlockSpec((1, _K, S), lambda b: (b, 0, 0)),
        out_shape=jax.ShapeDtypeStruct((B, _K, S), jnp.int32),
    )


def _make_sc_gather(B, R, NR, C):
    rpw = NR // _NW
    rc = 128
    while rpw % rc:
        rc //= 2
    nchunks = rpw // rc
    assert nchunks % 2 == 0 and _NW % B == 0
    wpb = _NW // B
    rpb = NR // B
    mesh = plsc.VectorSubcoreMesh(core_axis_name="c", subcore_axis_name="s")

    @functools.partial(
        pl.kernel,
        out_type=jax.ShapeDtypeStruct((B, rpb, C), jnp.float32),
        mesh=mesh,
        compiler_params=pltpu.CompilerParams(use_tc_tiling_on_sc=False),
        scratch_types=[
            pltpu.VMEM((rpw,), jnp.int32),
            pltpu.VMEM((rc, C), jnp.float32),
            pltpu.VMEM((rc, C), jnp.float32),
            pltpu.SemaphoreType.DMA,
            pltpu.SemaphoreType.DMA,
        ],
    )
    def sck(table_hbm, idx_hbm, out_hbm, idx_v, rows0_v, rows1_v,
            sem0, sem1):
        wid = lax.axis_index("s") * _NC + lax.axis_index("c")
        base_r = wid * rpw
        bat = wid // wpb
        base_in_b = (wid % wpb) * rpw
        bufs = (rows0_v, rows1_v)
        sems = (sem0, sem1)

        pltpu.sync_copy(idx_hbm.at[pl.ds(base_r, rpw)], idx_v)
        pltpu.async_copy(
            table_hbm.at[idx_v.at[pl.ds(0, rc)]], rows0_v, sem0)

        def pair(i, carry):
            for j in range(2):
                cur = i * 2 + j
                nxt = cur + 1

                @pl.when(nxt < nchunks)
                def _():
                    pltpu.async_copy(
                        table_hbm.at[idx_v.at[pl.ds(nxt * rc, rc)]],
                        bufs[1 - j], sems[1 - j])

                pltpu.make_async_copy(
                    table_hbm.at[idx_v.at[pl.ds(cur * rc, rc)]],
                    bufs[j], sems[j]).wait()
                pltpu.sync_copy(
                    bufs[j],
                    out_hbm.at[bat, pl.ds(base_in_b + cur * rc, rc)])
            return carry

        lax.fori_loop(0, nchunks // 2, pair, 0)

    return sck


def _conv_body(S, C, gath_ref, cent_ref, w_ref, out_ref):
    gath = jnp.reshape(gath_ref[0], (_K, S, C))
    cent = cent_ref[0][:S, :]
    edge = jnp.concatenate([gath, gath - cent[None, :, :]], axis=-1)
    edge2 = jnp.reshape(edge, (_K * S, 2 * C))
    t = jnp.dot(edge2, w_ref[...], preferred_element_type=jnp.float32)
    t3 = jnp.reshape(t, (_K, S, t.shape[-1]))
    out_ref[0] = jax.nn.gelu(jnp.max(t3, axis=0))


def _make_conv(B, N, S, C, D):
    return pl.pallas_call(
        functools.partial(_conv_body, S, C),
        grid=(B,),
        in_specs=[
            pl.BlockSpec((1, _K * S, C), lambda b: (b, 0, 0)),
            pl.BlockSpec((1, N, C), lambda b: (b, 0, 0)),
            pl.BlockSpec((2 * C, D), lambda b: (0, 0)),
        ],
        out_specs=pl.BlockSpec((1, S, D), lambda b: (b, 0, 0)),
        out_shape=jax.ShapeDtypeStruct((B, S, D), jnp.float32),
    )


def _head_body(x_ref, wle_ref, out_ref):
    out_ref[...] = jax.nn.gelu(
        jnp.dot(x_ref[...], wle_ref[...], preferred_element_type=jnp.float32))


def _make_head(B, N):
    return pl.pallas_call(
        _head_body,
        grid=(B,),
        in_specs=[
            pl.BlockSpec((1, N, 3), lambda b: (b, 0, 0)),
            pl.BlockSpec((3, 32), lambda b: (0, 0)),
        ],
        out_specs=pl.BlockSpec((1, N, 32), lambda b: (b, 0, 0)),
        out_shape=jax.ShapeDtypeStruct((B, N, 32), jnp.float32),
    )


def _tail_body(B, S, f_ref, wlast_ref, wlin1_ref, bng_ref, bnb_ref,
               wlin2_ref, blin2_ref, out_ref):
    h = jax.nn.gelu(jnp.dot(f_ref[...], wlast_ref[...],
                            preferred_element_type=jnp.float32))
    h3 = jnp.reshape(h, (B, S, h.shape[-1]))
    hmax = jnp.max(h3, axis=1)
    havg = jnp.mean(h3, axis=1)
    g = jnp.concatenate([hmax, havg], axis=-1)
    g = jnp.dot(g, wlin1_ref[...], preferred_element_type=jnp.float32)
    g = g * bng_ref[...] + bnb_ref[...]
    g = jax.nn.gelu(g)
    out_ref[...] = (jnp.dot(g, wlin2_ref[...],
                            preferred_element_type=jnp.float32)
                    + blin2_ref[...])


def _make_tail(B, S):
    return pl.pallas_call(
        functools.partial(_tail_body, B, S),
        out_shape=jax.ShapeDtypeStruct((B, 40), jnp.float32),
    )


def kernel(x, W_le, W1, W2, W3, W4, W_last, W_lin1, bn_g, bn_b,
           W_lin2, b_lin2):
    B, N, _ = x.shape
    levels = [
        (N, N, 32, 64, W1),
        (N, N // 2, 64, 128, W2),
        (N // 2, N // 4, 128, 256, W3),
        (N // 4, N // 8, 256, 512, W4),
    ]
    xT = jnp.transpose(x, (0, 2, 1))
    idxs = [
        _make_topk(B, n, s)(x[:, :n, :], xT[:, :, :n])
        for (n, s, c, d, w) in levels
    ]

    feat = _make_head(B, N)(x, W_le)
    for li, (n, s, c, d, w) in enumerate(levels):
        gath = _make_sc_gather(B, B * n, B * s * _K, c)(
            jnp.reshape(feat, (B * n, c)),
            jnp.reshape(idxs[li], (B * s * _K,)))
        feat = _make_conv(B, n, s, c, d)(gath, feat, w)

    S4 = N // 8
    return _make_tail(B, S4)(
        jnp.reshape(feat, (B * S4, 512)),
        W_last, W_lin1,
        jnp.reshape(bn_g, (1, -1)), jnp.reshape(bn_b, (1, -1)),
        W_lin2, jnp.reshape(b_lin2, (1, -1)))


# speedup vs baseline: 15.0810x; 1.1458x over previous
import functools

import jax
import jax.numpy as jnp
from jax import lax
from jax.experimental import pallas as pl
from jax.experimental.pallas import tpu as pltpu
from jax.experimental.pallas import tpu_sc as plsc

_NC = 2
_NS = 16
_NW = _NC * _NS
_K = 16
_INF = 3.0e38

_LEVELS = [
    (1024, 1024, 32, 64),
    (1024, 512, 64, 128),
    (512, 256, 128, 256),
    (256, 128, 256, 512),
]


def _pad128(c):
    return max(128, c)


def _prep_body(N, x_ref, xT_ref, wle_ref, feat_ref, i1_ref, i2_ref, i3_ref,
               i4_ref):
    b = pl.program_id(0)
    xyz = x_ref[0]
    xyzT = xT_ref[0]

    feat_ref[0] = jax.nn.gelu(
        jnp.dot(xyz, wle_ref[...], preferred_element_type=jnp.float32))

    n_pt = jnp.sum(xyz * xyz, axis=1, keepdims=True)
    n_ct = jnp.sum(xyzT * xyzT, axis=0, keepdims=True)

    for (n, s, _, _), out_ref in zip(_LEVELS, (i1_ref, i2_ref, i3_ref,
                                               i4_ref)):
        cross = jnp.dot(xyz[:n, :], xyzT[:, :s],
                        preferred_element_type=jnp.float32)
        d2 = n_ct[:, :s] + n_pt[:n, :] - 2.0 * cross
        iota_n = lax.broadcasted_iota(jnp.int32, (n, s), 0)
        rows = []
        for _ in range(_K):
            m = jnp.min(d2, axis=0, keepdims=True)
            am = jnp.min(jnp.where(d2 <= m, iota_n, n), axis=0,
                         keepdims=True)
            rows.append(am)
            d2 = jnp.where(iota_n == am, _INF, d2)
        out_ref[0] = jnp.concatenate(rows, axis=0) + b * n


def _make_prep(B, N):
    return pl.pallas_call(
        functools.partial(_prep_body, N),
        grid=(B,),
        in_specs=[
            pl.BlockSpec((1, N, 3), lambda b: (b, 0, 0)),
            pl.BlockSpec((1, 3, N), lambda b: (b, 0, 0)),
            pl.BlockSpec((3, 128), lambda b: (0, 0)),
        ],
        out_specs=[
            pl.BlockSpec((1, N, 128), lambda b: (b, 0, 0)),
        ] + [
            pl.BlockSpec((1, _K, s), lambda b, _s=s: (b, 0, 0))
            for (_, s, _, _) in _LEVELS
        ],
        out_shape=[
            jax.ShapeDtypeStruct((B, N, 128), jnp.float32),
        ] + [
            jax.ShapeDtypeStruct((B, _K, s), jnp.int32)
            for (_, s, _, _) in _LEVELS
        ],
    )


def _make_sc_gather(B, R, NR, C):
    rpw = NR // _NW
    rc = 128
    while rpw % rc:
        rc //= 2
    nchunks = rpw // rc
    assert nchunks % 2 == 0 and _NW % B == 0 and C % 128 == 0
    wpb = _NW // B
    rpb = NR // B
    mesh = plsc.VectorSubcoreMesh(core_axis_name="c", subcore_axis_name="s")

    @functools.partial(
        pl.kernel,
        out_type=jax.ShapeDtypeStruct((B, rpb, C), jnp.float32),
        mesh=mesh,
        scratch_types=[
            pltpu.VMEM((rpw,), jnp.int32),
            pltpu.VMEM((rc, C), jnp.float32),
            pltpu.VMEM((rc, C), jnp.float32),
            pltpu.SemaphoreType.DMA,
            pltpu.SemaphoreType.DMA,
        ],
    )
    def sck(table_hbm, idx_hbm, out_hbm, idx_v, rows0_v, rows1_v,
            sem0, sem1):
        wid = lax.axis_index("s") * _NC + lax.axis_index("c")
        base_r = wid * rpw
        bat = wid // wpb
        base_in_b = (wid % wpb) * rpw
        bufs = (rows0_v, rows1_v)
        sems = (sem0, sem1)

        pltpu.sync_copy(idx_hbm.at[pl.ds(base_r, rpw)], idx_v)
        pltpu.async_copy(
            table_hbm.at[idx_v.at[pl.ds(0, rc)]], rows0_v, sem0)

        def pair(i, carry):
            for j in range(2):
                cur = i * 2 + j
                nxt = cur + 1

                @pl.when(nxt < nchunks)
                def _():
                    pltpu.async_copy(
                        table_hbm.at[idx_v.at[pl.ds(nxt * rc, rc)]],
                        bufs[1 - j], sems[1 - j])

                pltpu.make_async_copy(
                    table_hbm.at[idx_v.at[pl.ds(cur * rc, rc)]],
                    bufs[j], sems[j]).wait()
                pltpu.sync_copy(
                    bufs[j],
                    out_hbm.at[bat, pl.ds(base_in_b + cur * rc, rc)])
            return carry

        lax.fori_loop(0, nchunks // 2, pair, 0)

    return sck


def _conv_body(S, C, CT, gath_ref, cent_ref, w_ref, out_ref):
    gath = jnp.reshape(gath_ref[0], (_K, S, CT))[:, :, :C]
    cent = cent_ref[0][:S, :C]
    edge = jnp.concatenate([gath, gath - cent[None, :, :]], axis=-1)
    edge2 = jnp.reshape(edge, (_K * S, 2 * C))
    t = jnp.dot(edge2, w_ref[...], preferred_element_type=jnp.float32)
    t3 = jnp.reshape(t, (_K, S, t.shape[-1]))
    out_ref[0] = jax.nn.gelu(jnp.max(t3, axis=0))


def _make_conv(B, N, S, C, CT, D, DT):
    return pl.pallas_call(
        functools.partial(_conv_body, S, C, CT),
        grid=(B,),
        in_specs=[
            pl.BlockSpec((1, _K * S, CT), lambda b: (b, 0, 0)),
            pl.BlockSpec((1, N, CT), lambda b: (b, 0, 0)),
            pl.BlockSpec((2 * C, DT), lambda b: (0, 0)),
        ],
        out_specs=pl.BlockSpec((1, S, DT), lambda b: (b, 0, 0)),
        out_shape=jax.ShapeDtypeStruct((B, S, DT), jnp.float32),
    )


def _conv_tail_body(S, C, gath_ref, cent_ref, w_ref, wlast_ref, wlin1_ref,
                    bng_ref, bnb_ref, wlin2_ref, blin2_ref, out_ref):
    gath = jnp.reshape(gath_ref[0], (_K, S, C))
    cent = cent_ref[0][:S, :]
    edge = jnp.concatenate([gath, gath - cent[None, :, :]], axis=-1)
    edge2 = jnp.reshape(edge, (_K * S, 2 * C))
    t = jnp.dot(edge2, w_ref[...], preferred_element_type=jnp.float32)
    t3 = jnp.reshape(t, (_K, S, t.shape[-1]))
    f4 = jax.nn.gelu(jnp.max(t3, axis=0))

    h = jax.nn.gelu(jnp.dot(f4, wlast_ref[...],
                            preferred_element_type=jnp.float32))
    hmax = jnp.max(h, axis=0, keepdims=True)
    havg = jnp.mean(h, axis=0, keepdims=True)
    g = jnp.concatenate([hmax, havg], axis=-1)
    g = jnp.dot(g, wlin1_ref[...], preferred_element_type=jnp.float32)
    g = g * bng_ref[...] + bnb_ref[...]
    g = jax.nn.gelu(g)
    out_ref[0] = (jnp.dot(g, wlin2_ref[...],
                          preferred_element_type=jnp.float32)
                  + blin2_ref[...])


def _make_conv_tail(B, N, S, C):
    return pl.pallas_call(
        functools.partial(_conv_tail_body, S, C),
        grid=(B,),
        in_specs=[
            pl.BlockSpec((1, _K * S, C), lambda b: (b, 0, 0)),
            pl.BlockSpec((1, N, C), lambda b: (b, 0, 0)),
            pl.BlockSpec((2 * C, 512), lambda b: (0, 0)),
            pl.BlockSpec((512, 1024), lambda b: (0, 0)),
            pl.BlockSpec((2048, 512), lambda b: (0, 0)),
            pl.BlockSpec((1, 512), lambda b: (0, 0)),
            pl.BlockSpec((1, 512), lambda b: (0, 0)),
            pl.BlockSpec((512, 40), lambda b: (0, 0)),
            pl.BlockSpec((1, 40), lambda b: (0, 0)),
        ],
        out_specs=pl.BlockSpec((1, 1, 40), lambda b: (b, 0, 0)),
        out_shape=jax.ShapeDtypeStruct((B, 1, 40), jnp.float32),
    )


def kernel(x, W_le, W1, W2, W3, W4, W_last, W_lin1, bn_g, bn_b,
           W_lin2, b_lin2):
    B, N, _ = x.shape
    f32 = jnp.float32

    xT = jnp.transpose(x, (0, 2, 1))
    wle_p = jnp.zeros((3, 128), f32).at[:, :32].set(W_le)
    w1_p = jnp.zeros((64, 128), f32).at[:, :64].set(W1)

    feat, i1, i2, i3, i4 = _make_prep(B, N)(x, xT, wle_p)
    idxs = (i1, i2, i3, i4)
    ws = (w1_p, W2, W3, W4)

    for li, (n, s, c, d) in enumerate(_LEVELS[:3]):
        ct = _pad128(c)
        dt = _pad128(d)
        gath = _make_sc_gather(B, B * n, B * s * _K, ct)(
            jnp.reshape(feat, (B * n, ct)),
            jnp.reshape(idxs[li], (B * s * _K,)))
        feat = _make_conv(B, n, s, c, ct, d, dt)(gath, feat, ws[li])

    n, s, c, d = _LEVELS[3]
    gath = _make_sc_gather(B, B * n, B * s * _K, c)(
        jnp.reshape(feat, (B * n, c)),
        jnp.reshape(idxs[3], (B * s * _K,)))
    out = _make_conv_tail(B, n, s, c)(
        gath, feat, W4, W_last, W_lin1,
        jnp.reshape(bn_g, (1, -1)), jnp.reshape(bn_b, (1, -1)),
        W_lin2, jnp.reshape(b_lin2, (1, -1)))
    return jnp.reshape(out, (B, 40))


# speedup vs baseline: 18.1359x; 1.2026x over previous
import functools

import jax
import jax.numpy as jnp
from jax import lax
from jax.experimental import pallas as pl
from jax.experimental.pallas import tpu as pltpu
from jax.experimental.pallas import tpu_sc as plsc

_NC = 2
_NS = 16
_NW = _NC * _NS
_K = 16
_INF = 3.0e38

_LEVELS = [
    (1024, 1024, 32, 64),
    (1024, 512, 64, 128),
    (512, 256, 128, 256),
    (256, 128, 256, 512),
]


def _pad128(c):
    return max(128, c)


def _prep_body(N, x_ref, xT_ref, wle_ref, feat_ref, i1_ref, i3_ref,
               i4_ref):
    b = pl.program_id(0)
    xyz = x_ref[0]
    xyzT = xT_ref[0]

    feat_ref[0] = jax.nn.gelu(
        jnp.dot(xyz, wle_ref[...], preferred_element_type=jnp.float32))

    n_pt = jnp.sum(xyz * xyz, axis=1, keepdims=True)
    n_ct = jnp.sum(xyzT * xyzT, axis=0, keepdims=True)

    for (n, s, _, _), out_ref in zip(
            (_LEVELS[0], _LEVELS[2], _LEVELS[3]),
            (i1_ref, i3_ref, i4_ref)):
        cross = jnp.dot(xyz[:n, :], xyzT[:, :s],
                        preferred_element_type=jnp.float32)
        d2 = n_ct[:, :s] + n_pt[:n, :] - 2.0 * cross
        iota_n = lax.broadcasted_iota(jnp.int32, (n, s), 0)
        rows = []
        for _ in range(_K):
            m = jnp.min(d2, axis=0, keepdims=True)
            am = jnp.min(jnp.where(d2 <= m, iota_n, n), axis=0,
                         keepdims=True)
            rows.append(am)
            d2 = jnp.where(iota_n == am, _INF, d2)
        out_ref[0] = jnp.concatenate(rows, axis=0) + b * n


def _make_prep(B, N):
    return pl.pallas_call(
        functools.partial(_prep_body, N),
        grid=(B,),
        in_specs=[
            pl.BlockSpec((1, N, 3), lambda b: (b, 0, 0)),
            pl.BlockSpec((1, 3, N), lambda b: (b, 0, 0)),
            pl.BlockSpec((3, 128), lambda b: (0, 0)),
        ],
        out_specs=[
            pl.BlockSpec((1, N, 128), lambda b: (b, 0, 0)),
        ] + [
            pl.BlockSpec((1, _K, s), lambda b, _s=s: (b, 0, 0))
            for (_, s, _, _) in (_LEVELS[0], _LEVELS[2], _LEVELS[3])
        ],
        out_shape=[
            jax.ShapeDtypeStruct((B, N, 128), jnp.float32),
        ] + [
            jax.ShapeDtypeStruct((B, _K, s), jnp.int32)
            for (_, s, _, _) in (_LEVELS[0], _LEVELS[2], _LEVELS[3])
        ],
    )


def _make_sc_gather(B, R, NR, C):
    rpw = NR // _NW
    rc = 128
    while rpw % rc:
        rc //= 2
    nchunks = rpw // rc
    assert nchunks % 2 == 0 and _NW % B == 0 and C % 128 == 0
    wpb = _NW // B
    rpb = NR // B
    mesh = plsc.VectorSubcoreMesh(core_axis_name="c", subcore_axis_name="s")

    @functools.partial(
        pl.kernel,
        out_type=jax.ShapeDtypeStruct((B, rpb, C), jnp.float32),
        mesh=mesh,
        scratch_types=[
            pltpu.VMEM((rpw,), jnp.int32),
            pltpu.VMEM((rc, C), jnp.float32),
            pltpu.VMEM((rc, C), jnp.float32),
            pltpu.SemaphoreType.DMA,
            pltpu.SemaphoreType.DMA,
        ],
    )
    def sck(table_hbm, idx_hbm, out_hbm, idx_v, rows0_v, rows1_v,
            sem0, sem1):
        wid = lax.axis_index("s") * _NC + lax.axis_index("c")
        base_r = wid * rpw
        bat = wid // wpb
        base_in_b = (wid % wpb) * rpw
        bufs = (rows0_v, rows1_v)
        sems = (sem0, sem1)

        pltpu.sync_copy(idx_hbm.at[pl.ds(base_r, rpw)], idx_v)
        pltpu.async_copy(
            table_hbm.at[idx_v.at[pl.ds(0, rc)]], rows0_v, sem0)

        def pair(i, carry):
            for j in range(2):
                cur = i * 2 + j
                nxt = cur + 1

                @pl.when(nxt < nchunks)
                def _():
                    pltpu.async_copy(
                        table_hbm.at[idx_v.at[pl.ds(nxt * rc, rc)]],
                        bufs[1 - j], sems[1 - j])

                pltpu.make_async_copy(
                    table_hbm.at[idx_v.at[pl.ds(cur * rc, rc)]],
                    bufs[j], sems[j]).wait()
                pltpu.sync_copy(
                    bufs[j],
                    out_hbm.at[bat, pl.ds(base_in_b + cur * rc, rc)])
            return carry

        lax.fori_loop(0, nchunks // 2, pair, 0)

    return sck


def _conv_body(S, C, CT, gath_ref, cent_ref, w_ref, out_ref):
    gath = jnp.reshape(gath_ref[0], (_K, S, CT))[:, :, :C]
    cent = cent_ref[0][:S, :C]
    edge = jnp.concatenate([gath, gath - cent[None, :, :]], axis=-1)
    edge2 = jnp.reshape(edge, (_K * S, 2 * C))
    t = jnp.dot(edge2, w_ref[...], preferred_element_type=jnp.float32)
    t3 = jnp.reshape(t, (_K, S, t.shape[-1]))
    out_ref[0] = jax.nn.gelu(jnp.max(t3, axis=0))


def _make_conv(B, N, S, C, CT, D, DT):
    return pl.pallas_call(
        functools.partial(_conv_body, S, C, CT),
        grid=(B,),
        in_specs=[
            pl.BlockSpec((1, _K * S, CT), lambda b: (b, 0, 0)),
            pl.BlockSpec((1, N, CT), lambda b: (b, 0, 0)),
            pl.BlockSpec((2 * C, DT), lambda b: (0, 0)),
        ],
        out_specs=pl.BlockSpec((1, S, DT), lambda b: (b, 0, 0)),
        out_shape=jax.ShapeDtypeStruct((B, S, DT), jnp.float32),
    )


def _conv_tail_body(S, C, gath_ref, cent_ref, w_ref, wlast_ref, wlin1_ref,
                    bng_ref, bnb_ref, wlin2_ref, blin2_ref, out_ref):
    gath = jnp.reshape(gath_ref[0], (_K, S, C))
    cent = cent_ref[0][:S, :]
    edge = jnp.concatenate([gath, gath - cent[None, :, :]], axis=-1)
    edge2 = jnp.reshape(edge, (_K * S, 2 * C))
    t = jnp.dot(edge2, w_ref[...], preferred_element_type=jnp.float32)
    t3 = jnp.reshape(t, (_K, S, t.shape[-1]))
    f4 = jax.nn.gelu(jnp.max(t3, axis=0))

    h = jax.nn.gelu(jnp.dot(f4, wlast_ref[...],
                            preferred_element_type=jnp.float32))
    hmax = jnp.max(h, axis=0, keepdims=True)
    havg = jnp.mean(h, axis=0, keepdims=True)
    g = jnp.concatenate([hmax, havg], axis=-1)
    g = jnp.dot(g, wlin1_ref[...], preferred_element_type=jnp.float32)
    g = g * bng_ref[...] + bnb_ref[...]
    g = jax.nn.gelu(g)
    out_ref[0] = (jnp.dot(g, wlin2_ref[...],
                          preferred_element_type=jnp.float32)
                  + blin2_ref[...])


def _make_conv_tail(B, N, S, C):
    return pl.pallas_call(
        functools.partial(_conv_tail_body, S, C),
        grid=(B,),
        in_specs=[
            pl.BlockSpec((1, _K * S, C), lambda b: (b, 0, 0)),
            pl.BlockSpec((1, N, C), lambda b: (b, 0, 0)),
            pl.BlockSpec((2 * C, 512), lambda b: (0, 0)),
            pl.BlockSpec((512, 1024), lambda b: (0, 0)),
            pl.BlockSpec((2048, 512), lambda b: (0, 0)),
            pl.BlockSpec((1, 512), lambda b: (0, 0)),
            pl.BlockSpec((1, 512), lambda b: (0, 0)),
            pl.BlockSpec((512, 40), lambda b: (0, 0)),
            pl.BlockSpec((1, 40), lambda b: (0, 0)),
        ],
        out_specs=pl.BlockSpec((1, 1, 40), lambda b: (b, 0, 0)),
        out_shape=jax.ShapeDtypeStruct((B, 1, 40), jnp.float32),
    )


def kernel(x, W_le, W1, W2, W3, W4, W_last, W_lin1, bn_g, bn_b,
           W_lin2, b_lin2):
    B, N, _ = x.shape
    f32 = jnp.float32

    xT = jnp.transpose(x, (0, 2, 1))
    wle_p = jnp.zeros((3, 128), f32).at[:, :32].set(W_le)
    w1_p = jnp.zeros((64, 128), f32).at[:, :64].set(W1)

    feat, i1, i3, i4 = _make_prep(B, N)(x, xT, wle_p)
    i2 = i1[:, :, :512]
    idxs = (i1, i2, i3, i4)
    ws = (w1_p, W2, W3, W4)

    for li, (n, s, c, d) in enumerate(_LEVELS[:3]):
        ct = _pad128(c)
        dt = _pad128(d)
        gath = _make_sc_gather(B, B * n, B * s * _K, ct)(
            jnp.reshape(feat, (B * n, ct)),
            jnp.reshape(idxs[li], (B * s * _K,)))
        feat = _make_conv(B, n, s, c, ct, d, dt)(gath, feat, ws[li])

    n, s, c, d = _LEVELS[3]
    gath = _make_sc_gather(B, B * n, B * s * _K, c)(
        jnp.reshape(feat, (B * n, c)),
        jnp.reshape(idxs[3], (B * s * _K,)))
    out = _make_conv_tail(B, n, s, c)(
        gath, feat, W4, W_last, W_lin1,
        jnp.reshape(bn_g, (1, -1)), jnp.reshape(bn_b, (1, -1)),
        W_lin2, jnp.reshape(b_lin2, (1, -1)))
    return jnp.reshape(out, (B, 40))
